# Initial kernel scaffold; baseline (speedup 1.0000x reference)
import jax
import jax.numpy as jnp
from jax.experimental import pallas as pl


def kernel(x, edge_index, edge_weight, W1, b1, W2, b2):
    raise NotImplementedError("write your pallas kernel here")


# speedup vs baseline: 26.7759x; 26.7759x over previous
import functools

import jax
import jax.numpy as jnp
from jax import lax
from jax.experimental import pallas as pl
from jax.experimental.pallas import tpu as pltpu
from jax.experimental.pallas import tpu_sc as plsc

N = 10000
E = 320000
D_IN = 128
D_HID = 16
N_CLS = 40

NC = 2
NS = 16
NW = NC * NS
EPT = E // NW
SUB = 80
BIG = 2000
NSUB_BIG = BIG // SUB
NBIG = EPT // BIG
NSUB = EPT // SUB
RPT = N // NS

_mesh = plsc.VectorSubcoreMesh(core_axis_name="c", subcore_axis_name="s")


@functools.partial(
    pl.kernel,
    out_type=jax.ShapeDtypeStruct((NW, N), jnp.float32),
    mesh=_mesh,
    compiler_params=pltpu.CompilerParams(needs_layout_passes=False, use_tc_tiling_on_sc=False),
    scratch_types=[
        pltpu.VMEM((EPT,), jnp.int32),
        pltpu.VMEM((EPT,), jnp.float32),
        pltpu.VMEM((N,), jnp.float32),
    ],
)
def _deg_kernel(dst_hbm, ew_hbm, out_hbm, dst_v, ew_v, deg_v):
    c = lax.axis_index("c")
    s = lax.axis_index("s")
    wid = c * NS + s
    pltpu.sync_copy(dst_hbm.at[wid], dst_v)
    pltpu.sync_copy(ew_hbm.at[wid], ew_v)

    def zero_body(i, carry):
        deg_v[pl.ds(i * 16, 16)] = jnp.zeros((16,), jnp.float32)
        return carry

    lax.fori_loop(0, N // 16, zero_body, 0, unroll=4)

    def body(g, carry):
        idx = dst_v[pl.ds(g * 16, 16)]
        w = ew_v[pl.ds(g * 16, 16)]
        plsc.addupdate_scatter(deg_v, [idx], w)
        return carry

    lax.fori_loop(0, EPT // 16, body, 0, unroll=4)
    pltpu.sync_copy(deg_v, out_hbm.at[wid])


@functools.partial(
    pl.kernel,
    out_type=jax.ShapeDtypeStruct((NC, N, D_HID), jnp.float32),
    mesh=_mesh,
    compiler_params=pltpu.CompilerParams(needs_layout_passes=False, use_tc_tiling_on_sc=False),
    scratch_types=[
        pltpu.VMEM((NSUB_BIG, SUB), jnp.int32),
        pltpu.VMEM((NSUB_BIG, SUB), jnp.int32),
        pltpu.VMEM((NSUB_BIG, SUB), jnp.float32),
        pltpu.VMEM((BIG, D_HID), jnp.float32),
        pltpu.VMEM((RPT, D_HID), jnp.float32),
        pltpu.VMEM_SHARED((N, D_HID), jnp.float32),
        pltpu.SemaphoreType.DMA,
    ],
)
def _agg_kernel(hs_hbm, src_hbm, dst_hbm, ew_hbm, out_hbm,
                src_v, dst_v, ew_v, rows_v, zbuf_v, acc_s, sem):
    c = lax.axis_index("c")
    s = lax.axis_index("s")
    wid = c * NS + s

    def zero_body(i, carry):
        zbuf_v[i, :] = jnp.zeros((D_HID,), jnp.float32)
        return carry

    lax.fori_loop(0, RPT, zero_body, 0, unroll=8)
    pltpu.sync_copy(zbuf_v, acc_s.at[pl.ds(s * RPT, RPT)])
    plsc.subcore_barrier()

    lane = lax.iota(jnp.int32, 16)
    for b in range(NBIG):
        row0 = wid * NSUB + b * NSUB_BIG
        pltpu.sync_copy(src_hbm.at[pl.ds(row0, NSUB_BIG)], src_v)
        pltpu.sync_copy(dst_hbm.at[pl.ds(row0, NSUB_BIG)], dst_v)
        pltpu.sync_copy(ew_hbm.at[pl.ds(row0, NSUB_BIG)], ew_v)
        gathers = [
            pltpu.async_copy(hs_hbm.at[src_v.at[j]],
                             rows_v.at[pl.ds(j * SUB, SUB)], sem)
            for j in range(NSUB_BIG)
        ]
        for g in gathers:
            g.wait()

        def scale_body(j, carry):
            for q in range(SUB // 16):
                w16 = ew_v[j, pl.ds(q * 16, 16)]
                row_idx = j * SUB + q * 16 + lane
                for d in range(D_HID):
                    dvec = jnp.full((16,), d, jnp.int32)
                    col = plsc.load_gather(rows_v, [row_idx, dvec])
                    plsc.store_scatter(rows_v, [row_idx, dvec], col * w16)
            return carry

        lax.fori_loop(0, NSUB_BIG, scale_body, 0)

        scatters = [
            pltpu.async_copy(rows_v.at[pl.ds(j * SUB, SUB)],
                             acc_s.at[dst_v.at[j]], sem, add=True)
            for j in range(NSUB_BIG)
        ]
        for sc in scatters:
            sc.wait()

    plsc.subcore_barrier()
    pltpu.sync_copy(acc_s.at[pl.ds(s * RPT, RPT)], zbuf_v)
    pltpu.sync_copy(zbuf_v, out_hbm.at[c, pl.ds(s * RPT, RPT)])


_BLK = 1000
_GRID = N // _BLK


def _dinv_bc_body(degp_ref, out_ref):
    deg = 1.0 + jnp.sum(degp_ref[...], axis=0)
    di = lax.rsqrt(deg)
    out_ref[...] = jnp.broadcast_to(di[:, None], (N, D_HID))


@jax.jit
def _dinv_bc(degp):
    return pl.pallas_call(
        _dinv_bc_body,
        out_shape=jax.ShapeDtypeStruct((N, D_HID), jnp.float32),
    )(degp)


def _mm_scale_body(x_ref, w_ref, di_ref, hs_ref):
    h = jnp.dot(x_ref[...], w_ref[...], preferred_element_type=jnp.float32)
    hs_ref[...] = h * di_ref[...]


@jax.jit
def _mm_scale(x, W1, dinvb):
    return pl.pallas_call(
        _mm_scale_body,
        grid=(_GRID,),
        in_specs=[
            pl.BlockSpec((_BLK, D_IN), lambda i: (i, 0)),
            pl.BlockSpec((D_IN, D_HID), lambda i: (0, 0)),
            pl.BlockSpec((_BLK, D_HID), lambda i: (i, 0)),
        ],
        out_specs=pl.BlockSpec((_BLK, D_HID), lambda i: (i, 0)),
        out_shape=jax.ShapeDtypeStruct((N, D_HID), jnp.float32),
    )(x, W1, dinvb)


def _post1_body(p_ref, hs_ref, di_ref, b_ref, out_ref):
    di = di_ref[...]
    t = p_ref[0] + p_ref[1] + hs_ref[...]
    out1 = jax.nn.relu(di * t + b_ref[...])
    out_ref[...] = di * out1


@jax.jit
def _post1(parts, hs1, dinvb, b1r):
    return pl.pallas_call(
        _post1_body,
        grid=(_GRID,),
        in_specs=[
            pl.BlockSpec((NC, _BLK, D_HID), lambda i: (0, i, 0)),
            pl.BlockSpec((_BLK, D_HID), lambda i: (i, 0)),
            pl.BlockSpec((_BLK, D_HID), lambda i: (i, 0)),
            pl.BlockSpec((1, D_HID), lambda i: (0, 0)),
        ],
        out_specs=pl.BlockSpec((_BLK, D_HID), lambda i: (i, 0)),
        out_shape=jax.ShapeDtypeStruct((N, D_HID), jnp.float32),
    )(parts, hs1, dinvb, b1r)


def _final_body(p_ref, hs_ref, di_ref, w_ref, b_ref, out_ref):
    agg = di_ref[...] * (p_ref[0] + p_ref[1] + hs_ref[...])
    logits = jnp.dot(agg, w_ref[...], preferred_element_type=jnp.float32)
    logits = logits + b_ref[...]
    z = logits - jnp.max(logits, axis=1, keepdims=True)
    out_ref[...] = z - jnp.log(jnp.sum(jnp.exp(z), axis=1, keepdims=True))


@jax.jit
def _final(parts, hs2, dinvb, W2, b2r):
    return pl.pallas_call(
        _final_body,
        grid=(_GRID,),
        in_specs=[
            pl.BlockSpec((NC, _BLK, D_HID), lambda i: (0, i, 0)),
            pl.BlockSpec((_BLK, D_HID), lambda i: (i, 0)),
            pl.BlockSpec((_BLK, D_HID), lambda i: (i, 0)),
            pl.BlockSpec((D_HID, N_CLS), lambda i: (0, 0)),
            pl.BlockSpec((1, N_CLS), lambda i: (0, 0)),
        ],
        out_specs=pl.BlockSpec((_BLK, N_CLS), lambda i: (i, 0)),
        out_shape=jax.ShapeDtypeStruct((N, N_CLS), jnp.float32),
    )(parts, hs2, dinvb, W2, b2r)


def kernel(x, edge_index, edge_weight, W1, b1, W2, b2):
    src = edge_index[0]
    dst = edge_index[1]
    degp = _deg_kernel(dst.reshape(NW, EPT), edge_weight.reshape(NW, EPT))
    dinvb = _dinv_bc(degp)
    hs1 = _mm_scale(x, W1, dinvb)
    srcg = src.reshape(NW * NSUB, SUB)
    dstg = dst.reshape(NW * NSUB, SUB)
    ewg = edge_weight.reshape(NW * NSUB, SUB)
    parts1 = _agg_kernel(hs1, srcg, dstg, ewg)
    hs2 = _post1(parts1, hs1, dinvb, b1.reshape(1, D_HID))
    parts2 = _agg_kernel(hs2, srcg, dstg, ewg)
    return _final(parts2, hs2, dinvb, W2, b2.reshape(1, N_CLS))


# speedup vs baseline: 37.9112x; 1.4159x over previous
import functools

import jax
import jax.numpy as jnp
from jax import lax
from jax.experimental import pallas as pl
from jax.experimental.pallas import tpu as pltpu
from jax.experimental.pallas import tpu_sc as plsc

N = 10000
E = 320000
D_IN = 128
D_HID = 16
N_CLS = 40

NC = 2
NS = 16
NW = NC * NS
EPT = E // NW

GD = 4
NP = NW // GD
EPP = E // NP
CH = 4000
NCH = EPP // CH

_mesh = plsc.VectorSubcoreMesh(core_axis_name="c", subcore_axis_name="s")
_sc_params = pltpu.CompilerParams(
    needs_layout_passes=False, use_tc_tiling_on_sc=False)


@functools.partial(
    pl.kernel,
    out_type=jax.ShapeDtypeStruct((NW, N), jnp.float32),
    mesh=_mesh,
    compiler_params=_sc_params,
    scratch_types=[
        pltpu.VMEM((EPT,), jnp.int32),
        pltpu.VMEM((EPT,), jnp.float32),
        pltpu.VMEM((N,), jnp.float32),
    ],
)
def _deg_kernel(dst_hbm, ew_hbm, out_hbm, dst_v, ew_v, deg_v):
    c = lax.axis_index("c")
    s = lax.axis_index("s")
    wid = c * NS + s
    pltpu.sync_copy(dst_hbm.at[wid], dst_v)
    pltpu.sync_copy(ew_hbm.at[wid], ew_v)

    def zero_body(i, carry):
        deg_v[pl.ds(i * 16, 16)] = jnp.zeros((16,), jnp.float32)
        return carry

    lax.fori_loop(0, N // 16, zero_body, 0, unroll=4)

    def body(g, carry):
        idx = dst_v[pl.ds(g * 16, 16)]
        w = ew_v[pl.ds(g * 16, 16)]
        plsc.addupdate_scatter(deg_v, [idx], w)
        return carry

    lax.fori_loop(0, EPT // 16, body, 0, unroll=4)
    pltpu.sync_copy(deg_v, out_hbm.at[wid])


@functools.partial(
    pl.kernel,
    out_type=jax.ShapeDtypeStruct((NW, GD, N), jnp.float32),
    mesh=_mesh,
    compiler_params=_sc_params,
    scratch_types=(
        [pltpu.VMEM((N,), jnp.float32) for _ in range(GD)]
        + [pltpu.VMEM((N,), jnp.float32) for _ in range(GD)]
        + [
            pltpu.VMEM((2, CH), jnp.int32),
            pltpu.VMEM((2, CH), jnp.int32),
            pltpu.VMEM((2, CH), jnp.float32),
            pltpu.SemaphoreType.DMA,
        ]
    ),
)
def _agg_kernel(hsT_hbm, src_hbm, dst_hbm, ew_hbm, out_hbm,
                t0, t1, t2, t3, a0, a1, a2, a3, sb, db, wb, sem):
    c = lax.axis_index("c")
    s = lax.axis_index("s")
    wid = c * NS + s
    g = lax.rem(wid, GD)
    p = lax.div(wid, GD)
    tabs = (t0, t1, t2, t3)
    accs = (a0, a1, a2, a3)

    tab_copies = [
        pltpu.async_copy(hsT_hbm.at[g * GD + d], tabs[d], sem)
        for d in range(GD)
    ]

    def zero_body(i, carry):
        z = jnp.zeros((16,), jnp.float32)
        for d in range(GD):
            accs[d][pl.ds(i * 16, 16)] = z
        return carry

    lax.fori_loop(0, N // 16, zero_body, 0, unroll=2)
    for cp in tab_copies:
        cp.wait()

    eb = p * EPP
    pend = [
        pltpu.async_copy(src_hbm.at[pl.ds(eb, CH)], sb.at[0], sem),
        pltpu.async_copy(dst_hbm.at[pl.ds(eb, CH)], db.at[0], sem),
        pltpu.async_copy(ew_hbm.at[pl.ds(eb, CH)], wb.at[0], sem),
    ]
    for ch in range(NCH):
        buf = ch % 2
        for cp in pend:
            cp.wait()
        if ch + 1 < NCH:
            nb = eb + (ch + 1) * CH
            pend = [
                pltpu.async_copy(src_hbm.at[pl.ds(nb, CH)], sb.at[1 - buf], sem),
                pltpu.async_copy(dst_hbm.at[pl.ds(nb, CH)], db.at[1 - buf], sem),
                pltpu.async_copy(ew_hbm.at[pl.ds(nb, CH)], wb.at[1 - buf], sem),
            ]

        def body(i, carry):
            base = i * 16
            sv = sb[buf, pl.ds(base, 16)]
            dv = db[buf, pl.ds(base, 16)]
            wv = wb[buf, pl.ds(base, 16)]
            for d in range(GD):
                col = plsc.load_gather(tabs[d], [sv])
                plsc.addupdate_scatter(accs[d], [dv], col * wv)
            return carry

        lax.fori_loop(0, CH // 16, body, 0, unroll=2)

    for d in range(GD):
        pltpu.sync_copy(accs[d], out_hbm.at[wid, d])


def _prep_body(degp_ref, x_ref, w_ref, hsT_ref, dinv_ref):
    deg = 1.0 + jnp.sum(degp_ref[...], axis=0)
    di = lax.rsqrt(deg)[None, :]
    dinv_ref[...] = di
    h = jnp.dot(x_ref[...], w_ref[...], preferred_element_type=jnp.float32)
    hsT_ref[...] = h.T * di


@jax.jit
def _prep(degp, x, W1):
    return pl.pallas_call(
        _prep_body,
        out_shape=(
            jax.ShapeDtypeStruct((D_HID, N), jnp.float32),
            jax.ShapeDtypeStruct((1, N), jnp.float32),
        ),
    )(degp, x, W1)


def _post1_body(p_ref, hsT_ref, dinv_ref, b_ref, out_ref):
    tmpT = jnp.sum(p_ref[...], axis=0).reshape(D_HID, N)
    di = dinv_ref[...]
    out1 = jax.nn.relu(di * (tmpT + hsT_ref[...]) + b_ref[...])
    out_ref[...] = di * out1


@jax.jit
def _post1(parts, hs1T, dinv, b1c):
    return pl.pallas_call(
        _post1_body,
        out_shape=jax.ShapeDtypeStruct((D_HID, N), jnp.float32),
    )(parts, hs1T, dinv, b1c)


def _final_body(p_ref, hsT_ref, dinv_ref, w_ref, b_ref, out_ref):
    tmpT = jnp.sum(p_ref[...], axis=0).reshape(D_HID, N)
    aggT = dinv_ref[...] * (tmpT + hsT_ref[...])
    logits = jnp.dot(aggT.T, w_ref[...], preferred_element_type=jnp.float32)
    logits = logits + b_ref[...]
    z = logits - jnp.max(logits, axis=1, keepdims=True)
    out_ref[...] = z - jnp.log(jnp.sum(jnp.exp(z), axis=1, keepdims=True))


@jax.jit
def _final(parts, hs2T, dinv, W2, b2r):
    return pl.pallas_call(
        _final_body,
        out_shape=jax.ShapeDtypeStruct((N, N_CLS), jnp.float32),
    )(parts, hs2T, dinv, W2, b2r)


def kernel(x, edge_index, edge_weight, W1, b1, W2, b2):
    src = edge_index[0]
    dst = edge_index[1]
    degp = _deg_kernel(dst.reshape(NW, EPT), edge_weight.reshape(NW, EPT))
    hs1T, dinv = _prep(degp, x, W1)
    parts1 = _agg_kernel(hs1T, src, dst, edge_weight)
    hs2T = _post1(parts1.reshape(NP, GD, GD, N), hs1T, dinv,
                  b1.reshape(D_HID, 1))
    parts2 = _agg_kernel(hs2T, src, dst, edge_weight)
    return _final(parts2.reshape(NP, GD, GD, N), hs2T, dinv, W2,
                  b2.reshape(1, N_CLS))
